# Initial kernel scaffold; baseline (speedup 1.0000x reference)
import functools

import jax
import jax.numpy as jnp
from jax import lax
from jax.experimental import pallas as pl
from jax.experimental.pallas import tpu as pltpu
from jax.experimental.pallas import tpu_sc as plsc

DIM = 128
GAMMA = 12.0
BATCH = 16384

NC = 2
NS = 16
L = 16
NW = NC * NS
B_PER_W = BATCH // NW
CH = 128
N_CH = B_PER_W // CH
KSEG = DIM // L


@functools.cache
def _build():
    mesh = plsc.VectorSubcoreMesh(
        core_axis_name="c", subcore_axis_name="s",
        num_cores=NC, num_subcores=NS,
    )

    @functools.partial(
        pl.kernel,
        mesh=mesh,
        out_type=jax.ShapeDtypeStruct((BATCH,), jnp.float32),
        scratch_types=[
            pltpu.VMEM((N_CH, CH), jnp.int32),
            pltpu.VMEM((N_CH, CH), jnp.int32),
            pltpu.VMEM((N_CH, CH), jnp.int32),
            pltpu.VMEM((CH, DIM), jnp.float32),
            pltpu.VMEM((CH, DIM), jnp.float32),
            pltpu.VMEM((CH, DIM), jnp.float32),
            pltpu.VMEM((L, L), jnp.float32),
            pltpu.VMEM((B_PER_W,), jnp.float32),
            pltpu.SemaphoreType.DMA,
        ],
    )
    def transe_kernel(hidx_hbm, ridx_hbm, tidx_hbm, ent_hbm, rel_hbm,
                      out_hbm, hi_v, ri_v, ti_v, h_v, r_v, t_v, tmp_v,
                      o_v, sem):
        w = lax.axis_index("s") * NC + lax.axis_index("c")
        base = w * B_PER_W

        pltpu.sync_copy(hidx_hbm.at[pl.ds(w * N_CH, N_CH)], hi_v)
        pltpu.sync_copy(ridx_hbm.at[pl.ds(w * N_CH, N_CH)], ri_v)
        pltpu.sync_copy(tidx_hbm.at[pl.ds(w * N_CH, N_CH)], ti_v)

        lane = lax.iota(jnp.int32, L)

        def do_chunk(j, carry):
            cp_h = pltpu.async_copy(ent_hbm.at[hi_v.at[j]], h_v, sem)
            cp_r = pltpu.async_copy(rel_hbm.at[ri_v.at[j]], r_v, sem)
            cp_t = pltpu.async_copy(ent_hbm.at[ti_v.at[j]], t_v, sem)
            cp_h.wait()
            cp_r.wait()
            cp_t.wait()

            def do_group(g, carry2):
                def do_row(i, carry3):
                    row = g * L + i
                    acc = jnp.zeros((L,), jnp.float32)
                    for k in range(KSEG):
                        hv = h_v[row, pl.ds(k * L, L)]
                        rv = r_v[row, pl.ds(k * L, L)]
                        tv = t_v[row, pl.ds(k * L, L)]
                        acc = acc + jnp.abs(hv + rv - tv)
                    tmp_v[i, :] = acc
                    return carry3

                lax.fori_loop(0, L, do_row, 0)
                svec = jnp.zeros((L,), jnp.float32)
                for k in range(L):
                    col = jnp.full((L,), k, jnp.int32)
                    svec = svec + plsc.load_gather(tmp_v, [lane, col])
                o_v[pl.ds(j * CH + g * L, L)] = GAMMA - svec
                return carry2

            lax.fori_loop(0, CH // L, do_group, 0)
            return carry

        lax.fori_loop(0, N_CH, do_chunk, 0)
        pltpu.sync_copy(o_v, out_hbm.at[pl.ds(base, B_PER_W)])

    return transe_kernel


def kernel(sample, entity_embedding, relation_embedding):
    hidx = sample[:, 0].reshape(NW * N_CH, CH)
    ridx = sample[:, 1].reshape(NW * N_CH, CH)
    tidx = sample[:, 2].reshape(NW * N_CH, CH)
    score = _build()(hidx, ridx, tidx, entity_embedding, relation_embedding)
    return score.reshape(BATCH, 1)


# speedup vs baseline: 1.6789x; 1.6789x over previous
import functools

import jax
import jax.numpy as jnp
from jax import lax
from jax.experimental import pallas as pl
from jax.experimental.pallas import tpu as pltpu
from jax.experimental.pallas import tpu_sc as plsc

DIM = 128
GAMMA = 12.0
BATCH = 16384

NC = 2
NS = 16
L = 16
NW = NC * NS
B_PER_W = BATCH // NW
CH = 128
N_CH = B_PER_W // CH
KSEG = DIM // L


@functools.cache
def _build():
    mesh = plsc.VectorSubcoreMesh(
        core_axis_name="c", subcore_axis_name="s",
        num_cores=NC, num_subcores=NS,
    )

    @functools.partial(
        pl.kernel,
        mesh=mesh,
        compiler_params=pltpu.CompilerParams(needs_layout_passes=False),
        out_type=jax.ShapeDtypeStruct((BATCH,), jnp.float32),
        scratch_types=[
            pltpu.VMEM((N_CH, CH), jnp.int32),
            pltpu.VMEM((N_CH, CH), jnp.int32),
            pltpu.VMEM((N_CH, CH), jnp.int32),
            pltpu.VMEM((CH, DIM), jnp.float32),
            pltpu.VMEM((CH, DIM), jnp.float32),
            pltpu.VMEM((CH, DIM), jnp.float32),
            pltpu.VMEM((B_PER_W,), jnp.float32),
            pltpu.SemaphoreType.DMA,
        ],
    )
    def transe_kernel(hidx_hbm, ridx_hbm, tidx_hbm, ent_hbm, rel_hbm,
                      out_hbm, hi_v, ri_v, ti_v, h_v, r_v, t_v,
                      o_v, sem):
        w = lax.axis_index("s") * NC + lax.axis_index("c")
        base = w * B_PER_W

        pltpu.sync_copy(hidx_hbm.at[pl.ds(w * N_CH, N_CH)], hi_v)
        pltpu.sync_copy(ridx_hbm.at[pl.ds(w * N_CH, N_CH)], ri_v)
        pltpu.sync_copy(tidx_hbm.at[pl.ds(w * N_CH, N_CH)], ti_v)

        lane = lax.iota(jnp.int32, L)

        def do_chunk(j, carry):
            cp_h = pltpu.async_copy(ent_hbm.at[hi_v.at[j]], h_v, sem)
            cp_r = pltpu.async_copy(rel_hbm.at[ri_v.at[j]], r_v, sem)
            cp_t = pltpu.async_copy(ent_hbm.at[ti_v.at[j]], t_v, sem)
            cp_h.wait()
            cp_r.wait()
            cp_t.wait()

            def do_group(g, carry2):
                def do_row(i, svec):
                    row = g * L + i
                    acc = jnp.zeros((L,), jnp.float32)
                    for k in range(KSEG):
                        hv = h_v[row, pl.ds(k * L, L)]
                        rv = r_v[row, pl.ds(k * L, L)]
                        tv = t_v[row, pl.ds(k * L, L)]
                        acc = acc + jnp.abs(hv + rv - tv)
                    s = jnp.sum(acc)
                    return svec + jnp.where(lane == i, s, 0.0)

                svec = lax.fori_loop(0, L, do_row, jnp.zeros((L,), jnp.float32))
                o_v[pl.ds(j * CH + g * L, L)] = GAMMA - svec
                return carry2

            lax.fori_loop(0, CH // L, do_group, 0)
            return carry

        lax.fori_loop(0, N_CH, do_chunk, 0)
        pltpu.sync_copy(o_v, out_hbm.at[pl.ds(base, B_PER_W)])

    return transe_kernel


def kernel(sample, entity_embedding, relation_embedding):
    hidx = sample[:, 0].reshape(NW * N_CH, CH)
    ridx = sample[:, 1].reshape(NW * N_CH, CH)
    tidx = sample[:, 2].reshape(NW * N_CH, CH)
    score = _build()(hidx, ridx, tidx, entity_embedding, relation_embedding)
    return score.reshape(BATCH, 1)


# speedup vs baseline: 1.8873x; 1.1241x over previous
import functools

import jax
import jax.numpy as jnp
from jax import lax
from jax.experimental import pallas as pl
from jax.experimental.pallas import tpu as pltpu
from jax.experimental.pallas import tpu_sc as plsc

DIM = 128
GAMMA = 12.0
BATCH = 16384

NC = 2
NS = 16
L = 16
NW = NC * NS
B_PER_W = BATCH // NW
CH = 128
N_CH = B_PER_W // CH
KSEG = DIM // L


@functools.cache
def _build():
    mesh = plsc.VectorSubcoreMesh(
        core_axis_name="c", subcore_axis_name="s",
        num_cores=NC, num_subcores=NS,
    )

    @functools.partial(
        pl.kernel,
        mesh=mesh,
        compiler_params=pltpu.CompilerParams(needs_layout_passes=False),
        out_type=jax.ShapeDtypeStruct((BATCH,), jnp.float32),
        scratch_types=[
            pltpu.VMEM((N_CH, CH), jnp.int32),
            pltpu.VMEM((N_CH, CH), jnp.int32),
            pltpu.VMEM((N_CH, CH), jnp.int32),
            pltpu.VMEM((2, CH, DIM), jnp.float32),
            pltpu.VMEM((2, CH, DIM), jnp.float32),
            pltpu.VMEM((2, CH, DIM), jnp.float32),
            pltpu.VMEM((B_PER_W,), jnp.float32),
            pltpu.SemaphoreType.DMA,
            pltpu.SemaphoreType.DMA,
        ],
    )
    def transe_kernel(hidx_hbm, ridx_hbm, tidx_hbm, ent_hbm, rel_hbm,
                      out_hbm, hi_v, ri_v, ti_v, h_v, r_v, t_v,
                      o_v, sem0, sem1):
        w = lax.axis_index("s") * NC + lax.axis_index("c")
        base = w * B_PER_W

        pltpu.sync_copy(hidx_hbm.at[pl.ds(w * N_CH, N_CH)], hi_v)
        pltpu.sync_copy(ridx_hbm.at[pl.ds(w * N_CH, N_CH)], ri_v)
        pltpu.sync_copy(tidx_hbm.at[pl.ds(w * N_CH, N_CH)], ti_v)

        lane = lax.iota(jnp.int32, L)
        sems = (sem0, sem1)

        def fire(j, b):
            return (
                pltpu.async_copy(ent_hbm.at[hi_v.at[j]], h_v.at[b], sems[b]),
                pltpu.async_copy(rel_hbm.at[ri_v.at[j]], r_v.at[b], sems[b]),
                pltpu.async_copy(ent_hbm.at[ti_v.at[j]], t_v.at[b], sems[b]),
            )

        def compute(j, b):
            hb, rb, tb = h_v.at[b], r_v.at[b], t_v.at[b]

            @plsc.parallel_loop(0, CH // L, 1)
            def _group(g):
                zero = jnp.zeros((L,), jnp.float32)

                @plsc.parallel_loop(0, L, 1, unroll=2, carry=zero)
                def svec(i, sv):
                    row = g * L + i
                    acc0 = jnp.zeros((L,), jnp.float32)
                    acc1 = jnp.zeros((L,), jnp.float32)
                    for k in range(0, KSEG, 2):
                        hv = hb[row, pl.ds(k * L, L)]
                        rv = rb[row, pl.ds(k * L, L)]
                        tv = tb[row, pl.ds(k * L, L)]
                        acc0 = acc0 + jnp.abs(hv + rv - tv)
                        hv = hb[row, pl.ds((k + 1) * L, L)]
                        rv = rb[row, pl.ds((k + 1) * L, L)]
                        tv = tb[row, pl.ds((k + 1) * L, L)]
                        acc1 = acc1 + jnp.abs(hv + rv - tv)
                    s = jnp.sum(acc0 + acc1)
                    return sv + jnp.where(lane == i, s, 0.0)

                o_v[pl.ds(j * CH + g * L, L)] = GAMMA - svec

        cps = fire(0, 0)
        for j in range(N_CH):
            b = j & 1
            for cp in cps:
                cp.wait()
            if j + 1 < N_CH:
                cps = fire(j + 1, 1 - b)
            compute(j, b)

        pltpu.sync_copy(o_v, out_hbm.at[pl.ds(base, B_PER_W)])

    return transe_kernel


def kernel(sample, entity_embedding, relation_embedding):
    hidx = sample[:, 0].reshape(NW * N_CH, CH)
    ridx = sample[:, 1].reshape(NW * N_CH, CH)
    tidx = sample[:, 2].reshape(NW * N_CH, CH)
    score = _build()(hidx, ridx, tidx, entity_embedding, relation_embedding)
    return score.reshape(BATCH, 1)


# speedup vs baseline: 1.9236x; 1.0192x over previous
import functools

import jax
import jax.numpy as jnp
from jax import lax
from jax.experimental import pallas as pl
from jax.experimental.pallas import tpu as pltpu
from jax.experimental.pallas import tpu_sc as plsc

DIM = 128
GAMMA = 12.0
BATCH = 16384

NC = 2
NS = 16
L = 16
NW = NC * NS
B_PER_W = BATCH // NW
CH = 128
N_CH = B_PER_W // CH
KSEG = DIM // L


@functools.cache
def _build():
    mesh = plsc.VectorSubcoreMesh(
        core_axis_name="c", subcore_axis_name="s",
        num_cores=NC, num_subcores=NS,
    )

    @functools.partial(
        pl.kernel,
        mesh=mesh,
        compiler_params=pltpu.CompilerParams(needs_layout_passes=False),
        out_type=jax.ShapeDtypeStruct((BATCH,), jnp.float32),
        scratch_types=[
            pltpu.VMEM((N_CH, CH), jnp.int32),
            pltpu.VMEM((N_CH, CH), jnp.int32),
            pltpu.VMEM((N_CH, CH), jnp.int32),
            pltpu.VMEM((2, CH, DIM), jnp.float32),
            pltpu.VMEM((2, CH, DIM), jnp.float32),
            pltpu.VMEM((2, CH, DIM), jnp.float32),
            pltpu.VMEM((B_PER_W,), jnp.float32),
            pltpu.SemaphoreType.DMA,
            pltpu.SemaphoreType.DMA,
        ],
    )
    def transe_kernel(hidx_hbm, ridx_hbm, tidx_hbm, ent_hbm, rel_hbm,
                      out_hbm, hi_v, ri_v, ti_v, h_v, r_v, t_v,
                      o_v, sem0, sem1):
        w = lax.axis_index("s") * NC + lax.axis_index("c")
        base = w * B_PER_W

        pltpu.sync_copy(hidx_hbm.at[pl.ds(w * N_CH, N_CH)], hi_v)
        pltpu.sync_copy(ridx_hbm.at[pl.ds(w * N_CH, N_CH)], ri_v)
        pltpu.sync_copy(tidx_hbm.at[pl.ds(w * N_CH, N_CH)], ti_v)

        lane = lax.iota(jnp.int32, L)
        sems = (sem0, sem1)

        def fire(j, b):
            pltpu.async_copy(ent_hbm.at[hi_v.at[j]], h_v.at[b], sems[b])
            pltpu.async_copy(rel_hbm.at[ri_v.at[j]], r_v.at[b], sems[b])
            pltpu.async_copy(ent_hbm.at[ti_v.at[j]], t_v.at[b], sems[b])

        def drain(j, b):
            pltpu.make_async_copy(ent_hbm.at[hi_v.at[j]], h_v.at[b],
                                  sems[b]).wait()
            pltpu.make_async_copy(rel_hbm.at[ri_v.at[j]], r_v.at[b],
                                  sems[b]).wait()
            pltpu.make_async_copy(ent_hbm.at[ti_v.at[j]], t_v.at[b],
                                  sems[b]).wait()

        def compute(j, b):
            hb, rb, tb = h_v.at[b], r_v.at[b], t_v.at[b]

            @plsc.parallel_loop(0, CH // L, 1)
            def _group(g):
                zero = jnp.zeros((L,), jnp.float32)

                @plsc.parallel_loop(0, L, 1, unroll=2, carry=zero)
                def svec(i, sv):
                    row = g * L + i
                    acc0 = jnp.zeros((L,), jnp.float32)
                    acc1 = jnp.zeros((L,), jnp.float32)
                    for k in range(0, KSEG, 2):
                        hv = hb[row, pl.ds(k * L, L)]
                        rv = rb[row, pl.ds(k * L, L)]
                        tv = tb[row, pl.ds(k * L, L)]
                        acc0 = acc0 + jnp.abs(hv + rv - tv)
                        hv = hb[row, pl.ds((k + 1) * L, L)]
                        rv = rb[row, pl.ds((k + 1) * L, L)]
                        tv = tb[row, pl.ds((k + 1) * L, L)]
                        acc1 = acc1 + jnp.abs(hv + rv - tv)
                    s = jnp.sum(acc0 + acc1)
                    return sv + jnp.where(lane == i, s, 0.0)

                o_v[pl.ds(j * CH + g * L, L)] = GAMMA - svec

        fire(0, 0)

        def do_pair(p, carry):
            j0 = 2 * p
            drain(j0, 0)
            fire(j0 + 1, 1)
            compute(j0, 0)
            drain(j0 + 1, 1)

            @pl.when(j0 + 2 < N_CH)
            def _():
                fire(j0 + 2, 0)

            compute(j0 + 1, 1)
            return carry

        lax.fori_loop(0, N_CH // 2, do_pair, 0)
        pltpu.sync_copy(o_v, out_hbm.at[pl.ds(base, B_PER_W)])

    return transe_kernel


def kernel(sample, entity_embedding, relation_embedding):
    hidx = sample[:, 0].reshape(NW * N_CH, CH)
    ridx = sample[:, 1].reshape(NW * N_CH, CH)
    tidx = sample[:, 2].reshape(NW * N_CH, CH)
    score = _build()(hidx, ridx, tidx, entity_embedding, relation_embedding)
    return score.reshape(BATCH, 1)


# speedup vs baseline: 1.9488x; 1.0131x over previous
import functools

import jax
import jax.numpy as jnp
from jax import lax
from jax.experimental import pallas as pl
from jax.experimental.pallas import tpu as pltpu
from jax.experimental.pallas import tpu_sc as plsc

DIM = 128
GAMMA = 12.0
BATCH = 16384

NC = 2
NS = 16
L = 16
NW = NC * NS
B_PER_W = BATCH // NW
CH = 128
N_CH = B_PER_W // CH
KSEG = DIM // L


@functools.cache
def _build():
    mesh = plsc.VectorSubcoreMesh(
        core_axis_name="c", subcore_axis_name="s",
        num_cores=NC, num_subcores=NS,
    )

    @functools.partial(
        pl.kernel,
        mesh=mesh,
        compiler_params=pltpu.CompilerParams(needs_layout_passes=False),
        out_type=jax.ShapeDtypeStruct((BATCH,), jnp.float32),
        scratch_types=[
            pltpu.VMEM((N_CH, CH), jnp.int32),
            pltpu.VMEM((N_CH, CH), jnp.int32),
            pltpu.VMEM((N_CH, CH), jnp.int32),
            pltpu.VMEM((2, CH, DIM), jnp.float32),
            pltpu.VMEM((2, CH, DIM), jnp.float32),
            pltpu.VMEM((2, CH, DIM), jnp.float32),
            pltpu.VMEM((B_PER_W,), jnp.float32),
            pltpu.SemaphoreType.DMA,
            pltpu.SemaphoreType.DMA,
        ],
    )
    def transe_kernel(hidx_hbm, ridx_hbm, tidx_hbm, ent_hbm, rel_hbm,
                      out_hbm, hi_v, ri_v, ti_v, h_v, r_v, t_v,
                      o_v, sem0, sem1):
        w = lax.axis_index("s") * NC + lax.axis_index("c")
        base = w * B_PER_W

        pltpu.sync_copy(hidx_hbm.at[pl.ds(w * N_CH, N_CH)], hi_v)
        pltpu.sync_copy(ridx_hbm.at[pl.ds(w * N_CH, N_CH)], ri_v)
        pltpu.sync_copy(tidx_hbm.at[pl.ds(w * N_CH, N_CH)], ti_v)

        lane = lax.iota(jnp.int32, L)
        sems = (sem0, sem1)

        def fire(j, b):
            pltpu.async_copy(ent_hbm.at[hi_v.at[j]], h_v.at[b], sems[b])
            pltpu.async_copy(rel_hbm.at[ri_v.at[j]], r_v.at[b], sems[b])
            pltpu.async_copy(ent_hbm.at[ti_v.at[j]], t_v.at[b], sems[b])

        def drain(j, b):
            pltpu.make_async_copy(ent_hbm.at[hi_v.at[j]], h_v.at[b],
                                  sems[b]).wait()
            pltpu.make_async_copy(rel_hbm.at[ri_v.at[j]], r_v.at[b],
                                  sems[b]).wait()
            pltpu.make_async_copy(ent_hbm.at[ti_v.at[j]], t_v.at[b],
                                  sems[b]).wait()

        def compute(j, b):
            hb, rb, tb = h_v.at[b], r_v.at[b], t_v.at[b]

            @plsc.parallel_loop(0, CH // L, 1)
            def _group(g):
                zero = jnp.zeros((L,), jnp.float32)

                @plsc.parallel_loop(0, L, 1, unroll=2, carry=zero)
                def svec(i, sv):
                    row = g * L + i
                    acc0 = jnp.zeros((L,), jnp.float32)
                    acc1 = jnp.zeros((L,), jnp.float32)
                    for k in range(0, KSEG, 2):
                        hv = hb[row, pl.ds(k * L, L)]
                        rv = rb[row, pl.ds(k * L, L)]
                        tv = tb[row, pl.ds(k * L, L)]
                        acc0 = acc0 + jnp.abs(hv + rv - tv)
                        hv = hb[row, pl.ds((k + 1) * L, L)]
                        rv = rb[row, pl.ds((k + 1) * L, L)]
                        tv = tb[row, pl.ds((k + 1) * L, L)]
                        acc1 = acc1 + jnp.abs(hv + rv - tv)
                    s = jnp.sum(acc0 + acc1)
                    return sv + jnp.where(lane == i, s, 0.0)

                o_v[pl.ds(j * CH + g * L, L)] = GAMMA - svec

        fire(0, 0)

        def do_chunk(j, carry):
            b = j % 2

            @pl.when(b == 0)
            def _():
                drain(j, 0)

                @pl.when(j + 1 < N_CH)
                def _():
                    fire(j + 1, 1)

            @pl.when(b == 1)
            def _():
                drain(j, 1)

                @pl.when(j + 1 < N_CH)
                def _():
                    fire(j + 1, 0)

            compute(j, b)
            return carry

        lax.fori_loop(0, N_CH, do_chunk, 0)
        pltpu.sync_copy(o_v, out_hbm.at[pl.ds(base, B_PER_W)])

    return transe_kernel


def kernel(sample, entity_embedding, relation_embedding):
    hidx = sample[:, 0].reshape(NW * N_CH, CH)
    ridx = sample[:, 1].reshape(NW * N_CH, CH)
    tidx = sample[:, 2].reshape(NW * N_CH, CH)
    score = _build()(hidx, ridx, tidx, entity_embedding, relation_embedding)
    return score.reshape(BATCH, 1)
